# baseline (speedup 1.0000x reference)
import functools

import jax
import jax.numpy as jnp
from jax.experimental import pallas as pl
from jax.experimental.pallas import tpu as pltpu


def _round_up(x, mult):
    return ((x + mult - 1) // mult) * mult


def _head_kernel(sub_ref, wsubt_ref, wfc_ref, bfc_ref, o_ref):
    h = jax.lax.dot_general(
        sub_ref[...], wsubt_ref[...],
        (((1,), (1,)), ((), ())),
        preferred_element_type=jnp.float32)
    o_ref[...] = (jnp.dot(h, wfc_ref[...],
                          preferred_element_type=jnp.float32) + bfc_ref[...])


_COL_TILE = 2048


def _fc_head(sub_id, w_sub, b_sub, w_fc, b_fc):
    _, n_sub = sub_id.shape
    hid = w_sub.shape[1]
    total = w_fc.shape[1]
    tn = min(total, _COL_TILE)
    nt = pl.cdiv(total, tn)
    sub_aug = jnp.concatenate(
        [sub_id, jnp.ones((1, 1), sub_id.dtype)], axis=1)
    wsub_aug_t = jnp.concatenate([w_sub, b_sub], axis=0).T
    return pl.pallas_call(
        _head_kernel,
        out_shape=jax.ShapeDtypeStruct((1, total), jnp.float32),
        grid=(nt,),
        in_specs=[
            pl.BlockSpec((1, n_sub + 1), lambda j: (0, 0)),
            pl.BlockSpec((hid, n_sub + 1), lambda j: (0, 0)),
            pl.BlockSpec((hid, tn), lambda j: (0, j)),
            pl.BlockSpec((1, tn), lambda j: (0, j)),
        ],
        out_specs=pl.BlockSpec((1, tn), lambda j: (0, j)),
        compiler_params=pltpu.CompilerParams(
            dimension_semantics=("parallel",)),
    )(sub_aug, wsub_aug_t, w_fc, b_fc)


_UNROLL = 16
_CHUNK = 192


def _gconv_kernel(x_ref, idx_ref, g_ref, be_ref, w_ref, b_ref, out_ref,
                  z3_ref, o3_ref, *, m, cin, cout, tm, slope, eps):
    x = x_ref[...]
    s = jnp.sum(x, axis=0, keepdims=True)
    q = jnp.sum(x * x, axis=0, keepdims=True)
    mean = s / m
    var = q / m - mean * mean
    inv = jax.lax.rsqrt(var + eps)
    scale = g_ref[...] * inv
    shift = be_ref[...] - mean * scale
    z = x * scale + shift
    y = jnp.where(z >= 0, z, slope * z)

    wstack = jnp.concatenate(
        [w_ref[cin * j:cin * (j + 1), :] for j in range(7)]
        + [jnp.zeros((cin, cout), jnp.float32)], axis=1)
    zz = jnp.dot(y, wstack, preferred_element_type=jnp.float32)
    gw = 8 * cout
    z3_ref[...] = zz.reshape(m, 1, gw)

    lane = jax.lax.broadcasted_iota(jnp.int32, (1, gw), 1)
    masks = [(lane // cout) == j for j in range(7)]
    bias = b_ref[...]

    def outer(ko, carry):
        tb = ko * (7 * _UNROLL)
        for ui in range(_UNROLL):
            rows = [z3_ref[idx_ref[0, 0, tb + 7 * ui + j]] for j in range(7)]
            sel = [jnp.where(masks[j], rows[j], 0.0) for j in range(7)]
            o3_ref[ko * _UNROLL + ui] = (((sel[0] + sel[1])
                                          + (sel[2] + sel[3]))
                                         + ((sel[4] + sel[5]) + sel[6]))
        return carry

    jax.lax.fori_loop(0, tm // _UNROLL, outer, 0)

    def red(kc, carry):
        blk = o3_ref[pl.ds(kc * _CHUNK, _CHUNK)]
        t2 = blk.reshape(_CHUNK, gw)
        t2 = t2 + pltpu.roll(t2, gw - 4 * cout, 1)
        t2 = t2 + pltpu.roll(t2, gw - 2 * cout, 1)
        t2 = t2 + pltpu.roll(t2, gw - cout, 1)
        out_ref[pl.ds(kc * _CHUNK, _CHUNK), :] = t2[:, :cout] + bias
        return carry

    jax.lax.fori_loop(0, tm // _CHUNK, red, 0)


def _gconv_layer(x, idx2, gamma, beta, w, b, *, slope=0.2, eps=1e-5):
    m, cin = x.shape
    cout = w.shape[1]
    tm = idx2.shape[2] // 7
    kern = functools.partial(_gconv_kernel, m=m, cin=cin, cout=cout,
                             tm=tm, slope=slope, eps=eps)
    return pl.pallas_call(
        kern,
        out_shape=jax.ShapeDtypeStruct((m, cout), jnp.float32),
        grid=(2,),
        in_specs=[
            pl.BlockSpec((m, cin), lambda c: (0, 0)),
            pl.BlockSpec((1, 1, idx2.shape[2]), lambda c: (c, 0, 0),
                         memory_space=pltpu.SMEM),
            pl.BlockSpec((1, cin), lambda c: (0, 0)),
            pl.BlockSpec((1, cin), lambda c: (0, 0)),
            pl.BlockSpec(w.shape, lambda c: (0, 0)),
            pl.BlockSpec((1, cout), lambda c: (0, 0)),
        ],
        out_specs=pl.BlockSpec((tm, cout), lambda c: (c, 0)),
        scratch_shapes=[
            pltpu.VMEM((m, 1, 8 * cout), jnp.float32),
            pltpu.VMEM((tm, 1, 8 * cout), jnp.float32),
        ],
        compiler_params=pltpu.CompilerParams(
            dimension_semantics=("parallel",)),
    )(x, idx2, gamma.reshape(1, cin), beta.reshape(1, cin), w, b)


def kernel(sub_id, neigh_orders, w_sub, b_sub, w_fc, b_fc,
           g0, be0, w0, b0, g1, be1, w1, b1, g2, be2, w2, b2):
    cin0 = 3
    n_vertex = w_fc.shape[1] // cin0

    x_flat = _fc_head(sub_id, w_sub, b_sub, w_fc, b_fc)
    x = x_flat.reshape(n_vertex, cin0)

    tm = _round_up((n_vertex + 1) // 2, _CHUNK)
    idx2 = jnp.pad(neigh_orders, (0, 2 * 7 * tm - neigh_orders.shape[0])
                   ).reshape(2, 1, 7 * tm)

    x1 = _gconv_layer(x, idx2, g0, be0, w0, b0)
    x2 = _gconv_layer(x1, idx2, g1, be1, w1, b1)
    x3 = _gconv_layer(x2, idx2, g2, be2, w2, b2)
    return x3


# speedup vs baseline: 1.4908x; 1.4908x over previous
import functools

import jax
import jax.numpy as jnp
from jax.experimental import pallas as pl
from jax.experimental.pallas import tpu as pltpu


def _round_up(x, mult):
    return ((x + mult - 1) // mult) * mult


def _head_kernel(sub_ref, wsubt_ref, wfc_ref, bfc_ref, o_ref):
    h = jax.lax.dot_general(
        sub_ref[...], wsubt_ref[...],
        (((1,), (1,)), ((), ())),
        preferred_element_type=jnp.float32)
    o_ref[...] = (jnp.dot(h, wfc_ref[...],
                          preferred_element_type=jnp.float32) + bfc_ref[...])


_COL_TILE = 2048


def _fc_head(sub_id, w_sub, b_sub, w_fc, b_fc):
    _, n_sub = sub_id.shape
    hid = w_sub.shape[1]
    total = w_fc.shape[1]
    tn = min(total, _COL_TILE)
    nt = pl.cdiv(total, tn)
    sub_aug = jnp.concatenate(
        [sub_id, jnp.ones((1, 1), sub_id.dtype)], axis=1)
    wsub_aug_t = jnp.concatenate([w_sub, b_sub], axis=0).T
    return pl.pallas_call(
        _head_kernel,
        out_shape=jax.ShapeDtypeStruct((1, total), jnp.float32),
        grid=(nt,),
        in_specs=[
            pl.BlockSpec((1, n_sub + 1), lambda j: (0, 0)),
            pl.BlockSpec((hid, n_sub + 1), lambda j: (0, 0)),
            pl.BlockSpec((hid, tn), lambda j: (0, j)),
            pl.BlockSpec((1, tn), lambda j: (0, j)),
        ],
        out_specs=pl.BlockSpec((1, tn), lambda j: (0, j)),
        compiler_params=pltpu.CompilerParams(
            dimension_semantics=("parallel",)),
    )(sub_aug, wsub_aug_t, w_fc, b_fc)


_UNROLL = 16
_CHUNK = 192


def _gconv_kernel(x_ref, idx_ref, g_ref, be_ref, w_ref, b_ref, out_ref,
                  z3_ref, o3_ref, *, m, cin, cout, tm, slope, eps):
    x = x_ref[...]
    s = jnp.sum(x, axis=0, keepdims=True)
    q = jnp.sum(x * x, axis=0, keepdims=True)
    mean = s / m
    var = q / m - mean * mean
    inv = jax.lax.rsqrt(var + eps)
    scale = g_ref[...] * inv
    shift = be_ref[...] - mean * scale
    z = x * scale + shift
    y = jnp.where(z >= 0, z, slope * z)

    wstack = jnp.concatenate(
        [w_ref[cin * j:cin * (j + 1), :] for j in range(7)]
        + [jnp.zeros((cin, cout), jnp.float32)], axis=1)
    zz = jnp.dot(y, wstack, preferred_element_type=jnp.float32)
    gw = 8 * cout
    z3_ref[...] = zz.reshape(m, 1, gw)

    lane = jax.lax.broadcasted_iota(jnp.int32, (1, gw), 1)
    masks = [(lane // cout) == j for j in range(7)]
    rid = jax.lax.broadcasted_iota(jnp.int32, (gw, cout), 0)
    cid = jax.lax.broadcasted_iota(jnp.int32, (gw, cout), 1)
    sel_r = ((rid % cout) == cid).astype(jnp.float32)
    bias = b_ref[...]

    def outer(ko, carry):
        tb = ko * (7 * _UNROLL)
        for ui in range(_UNROLL):
            rows = [z3_ref[idx_ref[0, 0, tb + 7 * ui + j]] for j in range(7)]
            sel = [jnp.where(masks[j], rows[j], 0.0) for j in range(7)]
            o3_ref[ko * _UNROLL + ui] = (((sel[0] + sel[1])
                                          + (sel[2] + sel[3]))
                                         + ((sel[4] + sel[5]) + sel[6]))
        return carry

    jax.lax.fori_loop(0, tm // _UNROLL, outer, 0)

    def red(kc, carry):
        blk = o3_ref[pl.ds(kc * _CHUNK, _CHUNK)]
        t2 = blk.reshape(_CHUNK, gw)
        o = jnp.dot(t2, sel_r, preferred_element_type=jnp.float32)
        out_ref[pl.ds(kc * _CHUNK, _CHUNK), :] = o + bias
        return carry

    jax.lax.fori_loop(0, tm // _CHUNK, red, 0)


def _gconv_layer(x, idx2, gamma, beta, w, b, *, slope=0.2, eps=1e-5):
    m, cin = x.shape
    cout = w.shape[1]
    tm = idx2.shape[2] // 7
    kern = functools.partial(_gconv_kernel, m=m, cin=cin, cout=cout,
                             tm=tm, slope=slope, eps=eps)
    return pl.pallas_call(
        kern,
        out_shape=jax.ShapeDtypeStruct((m, cout), jnp.float32),
        grid=(2,),
        in_specs=[
            pl.BlockSpec((m, cin), lambda c: (0, 0)),
            pl.BlockSpec((1, 1, idx2.shape[2]), lambda c: (c, 0, 0),
                         memory_space=pltpu.SMEM),
            pl.BlockSpec((1, cin), lambda c: (0, 0)),
            pl.BlockSpec((1, cin), lambda c: (0, 0)),
            pl.BlockSpec(w.shape, lambda c: (0, 0)),
            pl.BlockSpec((1, cout), lambda c: (0, 0)),
        ],
        out_specs=pl.BlockSpec((tm, cout), lambda c: (c, 0)),
        scratch_shapes=[
            pltpu.VMEM((m, 1, 8 * cout), jnp.float32),
            pltpu.VMEM((tm, 1, 8 * cout), jnp.float32),
        ],
        compiler_params=pltpu.CompilerParams(
            dimension_semantics=("parallel",)),
    )(x, idx2, gamma.reshape(1, cin), beta.reshape(1, cin), w, b)


def kernel(sub_id, neigh_orders, w_sub, b_sub, w_fc, b_fc,
           g0, be0, w0, b0, g1, be1, w1, b1, g2, be2, w2, b2):
    cin0 = 3
    n_vertex = w_fc.shape[1] // cin0

    x_flat = _fc_head(sub_id, w_sub, b_sub, w_fc, b_fc)
    x = x_flat.reshape(n_vertex, cin0)

    tm = _round_up((n_vertex + 1) // 2, _CHUNK)
    idx2 = jnp.pad(neigh_orders, (0, 2 * 7 * tm - neigh_orders.shape[0])
                   ).reshape(2, 1, 7 * tm)

    x1 = _gconv_layer(x, idx2, g0, be0, w0, b0)
    x2 = _gconv_layer(x1, idx2, g1, be1, w1, b1)
    x3 = _gconv_layer(x2, idx2, g2, be2, w2, b2)
    return x3


# speedup vs baseline: 1.5312x; 1.0271x over previous
import functools

import jax
import jax.numpy as jnp
from jax.experimental import pallas as pl
from jax.experimental.pallas import tpu as pltpu


def _round_up(x, mult):
    return ((x + mult - 1) // mult) * mult


def _head_kernel(sub_ref, wsubt_ref, wfc_ref, bfc_ref, o_ref):
    h = jax.lax.dot_general(
        sub_ref[...], wsubt_ref[...],
        (((1,), (1,)), ((), ())),
        preferred_element_type=jnp.float32)
    o_ref[...] = (jnp.dot(h, wfc_ref[...],
                          preferred_element_type=jnp.float32) + bfc_ref[...])


_COL_TILE = 2048


def _fc_head(sub_id, w_sub, b_sub, w_fc, b_fc):
    _, n_sub = sub_id.shape
    hid = w_sub.shape[1]
    total = w_fc.shape[1]
    tn = min(total, _COL_TILE)
    nt = pl.cdiv(total, tn)
    sub_aug = jnp.concatenate(
        [sub_id, jnp.ones((1, 1), sub_id.dtype)], axis=1)
    wsub_aug_t = jnp.concatenate([w_sub, b_sub], axis=0).T
    return pl.pallas_call(
        _head_kernel,
        out_shape=jax.ShapeDtypeStruct((1, total), jnp.float32),
        grid=(nt,),
        in_specs=[
            pl.BlockSpec((1, n_sub + 1), lambda j: (0, 0)),
            pl.BlockSpec((hid, n_sub + 1), lambda j: (0, 0)),
            pl.BlockSpec((hid, tn), lambda j: (0, j)),
            pl.BlockSpec((1, tn), lambda j: (0, j)),
        ],
        out_specs=pl.BlockSpec((1, tn), lambda j: (0, j)),
        compiler_params=pltpu.CompilerParams(
            dimension_semantics=("parallel",)),
    )(sub_aug, wsub_aug_t, w_fc, b_fc)


_UNROLL = 32
_CHUNK = 192


def _gconv_kernel(x_ref, idx_ref, g_ref, be_ref, w_ref, b_ref, out_ref,
                  z3_ref, o3_ref, *, m, cin, cout, tm, slope, eps):
    x = x_ref[...]
    s = jnp.sum(x, axis=0, keepdims=True)
    q = jnp.sum(x * x, axis=0, keepdims=True)
    mean = s / m
    var = q / m - mean * mean
    inv = jax.lax.rsqrt(var + eps)
    scale = g_ref[...] * inv
    shift = be_ref[...] - mean * scale
    z = x * scale + shift
    y = jnp.where(z >= 0, z, slope * z)

    wstack = jnp.concatenate(
        [w_ref[cin * j:cin * (j + 1), :] for j in range(7)]
        + [jnp.zeros((cin, cout), jnp.float32)], axis=1)
    zz = jnp.dot(y, wstack, preferred_element_type=jnp.float32)
    gw = 8 * cout
    z3_ref[...] = zz.reshape(m, 1, gw)

    lane = jax.lax.broadcasted_iota(jnp.int32, (1, gw), 1)
    masks = [(lane // cout) == j for j in range(7)]
    rid = jax.lax.broadcasted_iota(jnp.int32, (gw, cout), 0)
    cid = jax.lax.broadcasted_iota(jnp.int32, (gw, cout), 1)
    sel_r = ((rid % cout) == cid).astype(jnp.float32)
    bias = b_ref[...]

    def outer(ko, carry):
        tb = ko * (7 * _UNROLL)
        for ui in range(_UNROLL):
            rows = [z3_ref[idx_ref[0, 0, tb + 7 * ui + j]] for j in range(7)]
            sel = [jnp.where(masks[j], rows[j], 0.0) for j in range(7)]
            o3_ref[ko * _UNROLL + ui] = (((sel[0] + sel[1])
                                          + (sel[2] + sel[3]))
                                         + ((sel[4] + sel[5]) + sel[6]))
        return carry

    jax.lax.fori_loop(0, tm // _UNROLL, outer, 0)

    def red(kc, carry):
        blk = o3_ref[pl.ds(kc * _CHUNK, _CHUNK)]
        t2 = blk.reshape(_CHUNK, gw)
        o = jnp.dot(t2, sel_r, preferred_element_type=jnp.float32)
        out_ref[pl.ds(kc * _CHUNK, _CHUNK), :] = o + bias
        return carry

    jax.lax.fori_loop(0, tm // _CHUNK, red, 0)


def _gconv_layer(x, idx2, gamma, beta, w, b, *, slope=0.2, eps=1e-5):
    m, cin = x.shape
    cout = w.shape[1]
    tm = idx2.shape[2] // 7
    kern = functools.partial(_gconv_kernel, m=m, cin=cin, cout=cout,
                             tm=tm, slope=slope, eps=eps)
    return pl.pallas_call(
        kern,
        out_shape=jax.ShapeDtypeStruct((m, cout), jnp.float32),
        grid=(2,),
        in_specs=[
            pl.BlockSpec((m, cin), lambda c: (0, 0)),
            pl.BlockSpec((1, 1, idx2.shape[2]), lambda c: (c, 0, 0),
                         memory_space=pltpu.SMEM),
            pl.BlockSpec((1, cin), lambda c: (0, 0)),
            pl.BlockSpec((1, cin), lambda c: (0, 0)),
            pl.BlockSpec(w.shape, lambda c: (0, 0)),
            pl.BlockSpec((1, cout), lambda c: (0, 0)),
        ],
        out_specs=pl.BlockSpec((tm, cout), lambda c: (c, 0)),
        scratch_shapes=[
            pltpu.VMEM((m, 1, 8 * cout), jnp.float32),
            pltpu.VMEM((tm, 1, 8 * cout), jnp.float32),
        ],
        compiler_params=pltpu.CompilerParams(
            dimension_semantics=("parallel",)),
    )(x, idx2, gamma.reshape(1, cin), beta.reshape(1, cin), w, b)


def kernel(sub_id, neigh_orders, w_sub, b_sub, w_fc, b_fc,
           g0, be0, w0, b0, g1, be1, w1, b1, g2, be2, w2, b2):
    cin0 = 3
    n_vertex = w_fc.shape[1] // cin0

    x_flat = _fc_head(sub_id, w_sub, b_sub, w_fc, b_fc)
    x = x_flat.reshape(n_vertex, cin0)

    tm = _round_up((n_vertex + 1) // 2, _CHUNK)
    idx2 = jnp.pad(neigh_orders, (0, 2 * 7 * tm - neigh_orders.shape[0])
                   ).reshape(2, 1, 7 * tm)

    x1 = _gconv_layer(x, idx2, g0, be0, w0, b0)
    x2 = _gconv_layer(x1, idx2, g1, be1, w1, b1)
    x3 = _gconv_layer(x2, idx2, g2, be2, w2, b2)
    return x3


# speedup vs baseline: 1.5513x; 1.0132x over previous
import functools

import jax
import jax.numpy as jnp
from jax.experimental import pallas as pl
from jax.experimental.pallas import tpu as pltpu


def _round_up(x, mult):
    return ((x + mult - 1) // mult) * mult


def _head_kernel(sub_ref, wsubt_ref, wfc_ref, bfc_ref, o_ref):
    h = jax.lax.dot_general(
        sub_ref[...], wsubt_ref[...],
        (((1,), (1,)), ((), ())),
        preferred_element_type=jnp.float32)
    o_ref[...] = (jnp.dot(h, wfc_ref[...],
                          preferred_element_type=jnp.float32) + bfc_ref[...])


_COL_TILE = 4096


def _fc_head(sub_id, w_sub, b_sub, w_fc, b_fc):
    _, n_sub = sub_id.shape
    hid = w_sub.shape[1]
    total = w_fc.shape[1]
    tn = min(total, _COL_TILE)
    nt = pl.cdiv(total, tn)
    sub_aug = jnp.concatenate(
        [sub_id, jnp.ones((1, 1), sub_id.dtype)], axis=1)
    wsub_aug_t = jnp.concatenate([w_sub, b_sub], axis=0).T
    return pl.pallas_call(
        _head_kernel,
        out_shape=jax.ShapeDtypeStruct((1, total), jnp.float32),
        grid=(nt,),
        in_specs=[
            pl.BlockSpec((1, n_sub + 1), lambda j: (0, 0)),
            pl.BlockSpec((hid, n_sub + 1), lambda j: (0, 0)),
            pl.BlockSpec((hid, tn), lambda j: (0, j)),
            pl.BlockSpec((1, tn), lambda j: (0, j)),
        ],
        out_specs=pl.BlockSpec((1, tn), lambda j: (0, j)),
        compiler_params=pltpu.CompilerParams(
            dimension_semantics=("parallel",)),
    )(sub_aug, wsub_aug_t, w_fc, b_fc)


_UNROLL = 32
_CHUNK = 192


def _gconv_kernel(x_ref, idx_ref, g_ref, be_ref, w_ref, b_ref, out_ref,
                  z3_ref, o3_ref, *, m, cin, cout, tm, slope, eps):
    x = x_ref[...]
    s = jnp.sum(x, axis=0, keepdims=True)
    q = jnp.sum(x * x, axis=0, keepdims=True)
    mean = s / m
    var = q / m - mean * mean
    inv = jax.lax.rsqrt(var + eps)
    scale = g_ref[...] * inv
    shift = be_ref[...] - mean * scale
    z = x * scale + shift
    y = jnp.where(z >= 0, z, slope * z)

    wstack = jnp.concatenate(
        [w_ref[cin * j:cin * (j + 1), :] for j in range(7)]
        + [jnp.zeros((cin, cout), jnp.float32)], axis=1)
    zz = jnp.dot(y, wstack, preferred_element_type=jnp.float32)
    gw = 8 * cout
    z3_ref[...] = zz.reshape(m, 1, gw)

    lane = jax.lax.broadcasted_iota(jnp.int32, (1, gw), 1)
    masks = [(lane // cout) == j for j in range(7)]
    rid = jax.lax.broadcasted_iota(jnp.int32, (gw, cout), 0)
    cid = jax.lax.broadcasted_iota(jnp.int32, (gw, cout), 1)
    sel_r = ((rid % cout) == cid).astype(jnp.float32)
    bias = b_ref[...]

    def outer(ko, carry):
        tb = ko * (7 * _UNROLL)
        for ui in range(_UNROLL):
            rows = [z3_ref[idx_ref[0, 0, tb + 7 * ui + j]] for j in range(7)]
            sel = [jnp.where(masks[j], rows[j], 0.0) for j in range(7)]
            o3_ref[ko * _UNROLL + ui] = (((sel[0] + sel[1])
                                          + (sel[2] + sel[3]))
                                         + ((sel[4] + sel[5]) + sel[6]))
        return carry

    jax.lax.fori_loop(0, tm // _UNROLL, outer, 0)

    def red(kc, carry):
        blk = o3_ref[pl.ds(kc * _CHUNK, _CHUNK)]
        t2 = blk.reshape(_CHUNK, gw)
        o = jnp.dot(t2, sel_r, preferred_element_type=jnp.float32)
        out_ref[pl.ds(kc * _CHUNK, _CHUNK), :] = o + bias
        return carry

    jax.lax.fori_loop(0, tm // _CHUNK, red, 0)


def _gconv_layer(x, idx2, gamma, beta, w, b, *, slope=0.2, eps=1e-5):
    m, cin = x.shape
    cout = w.shape[1]
    tm = idx2.shape[2] // 7
    kern = functools.partial(_gconv_kernel, m=m, cin=cin, cout=cout,
                             tm=tm, slope=slope, eps=eps)
    return pl.pallas_call(
        kern,
        out_shape=jax.ShapeDtypeStruct((m, cout), jnp.float32),
        grid=(2,),
        in_specs=[
            pl.BlockSpec((m, cin), lambda c: (0, 0)),
            pl.BlockSpec((1, 1, idx2.shape[2]), lambda c: (c, 0, 0),
                         memory_space=pltpu.SMEM),
            pl.BlockSpec((1, cin), lambda c: (0, 0)),
            pl.BlockSpec((1, cin), lambda c: (0, 0)),
            pl.BlockSpec(w.shape, lambda c: (0, 0)),
            pl.BlockSpec((1, cout), lambda c: (0, 0)),
        ],
        out_specs=pl.BlockSpec((tm, cout), lambda c: (c, 0)),
        scratch_shapes=[
            pltpu.VMEM((m, 1, 8 * cout), jnp.float32),
            pltpu.VMEM((tm, 1, 8 * cout), jnp.float32),
        ],
        compiler_params=pltpu.CompilerParams(
            dimension_semantics=("parallel",)),
    )(x, idx2, gamma.reshape(1, cin), beta.reshape(1, cin), w, b)


def kernel(sub_id, neigh_orders, w_sub, b_sub, w_fc, b_fc,
           g0, be0, w0, b0, g1, be1, w1, b1, g2, be2, w2, b2):
    cin0 = 3
    n_vertex = w_fc.shape[1] // cin0

    x_flat = _fc_head(sub_id, w_sub, b_sub, w_fc, b_fc)
    x = x_flat.reshape(n_vertex, cin0)

    tm = _round_up((n_vertex + 1) // 2, _CHUNK)
    idx2 = jnp.pad(neigh_orders, (0, 2 * 7 * tm - neigh_orders.shape[0])
                   ).reshape(2, 1, 7 * tm)

    x1 = _gconv_layer(x, idx2, g0, be0, w0, b0)
    x2 = _gconv_layer(x1, idx2, g1, be1, w1, b1)
    x3 = _gconv_layer(x2, idx2, g2, be2, w2, b2)
    return x3
